# baseline (speedup 1.0000x reference)
import functools

import jax
import jax.numpy as jnp
from jax import lax
from jax.experimental import pallas as pl
from jax.experimental.pallas import tpu as pltpu
from jax.experimental.pallas import tpu_sc as plsc

B = 16384
D = 64
NW = 32
BPW = B // NW
L = 16
GROUPS = BPW // L
CHUNK = 128


def _rsqrt(y):
    i = plsc.bitcast(y, jnp.int32)
    i = jnp.int32(0x5F3759DF) - lax.shift_right_logical(i, 1)
    r = plsc.bitcast(i, jnp.float32)
    for _ in range(3):
        r = r * (jnp.float32(1.5) - jnp.float32(0.5) * y * r * r)
    return r


def _sc_partials(x, target, weight):
    mesh = plsc.VectorSubcoreMesh(core_axis_name="c", subcore_axis_name="s")

    @functools.partial(
        pl.kernel,
        mesh=mesh,
        out_type=jax.ShapeDtypeStruct((NW, L), jnp.float32),
        compiler_params=pltpu.CompilerParams(
            needs_layout_passes=False, use_tc_tiling_on_sc=False),
        scratch_types=[
            pltpu.VMEM((BPW,), jnp.int32),
            pltpu.VMEM((BPW, D), jnp.float32),
            pltpu.VMEM((BPW, D), jnp.float32),
            pltpu.VMEM((L,), jnp.float32),
            pltpu.SemaphoreType.DMA,
        ],
    )
    def k(x_hbm, tgt_hbm, w_hbm, out_hbm, idx_v, x_v, w_v, acc_v, sem):
        wid = lax.axis_index("s") * 2 + lax.axis_index("c")
        base = wid * BPW
        pltpu.sync_copy(tgt_hbm.at[pl.ds(base, BPW)], idx_v)
        pltpu.sync_copy(x_hbm.at[pl.ds(base, BPW)], x_v)
        copies = [
            pltpu.async_copy(
                w_hbm.at[idx_v.at[pl.ds(j * CHUNK, CHUNK)]],
                w_v.at[pl.ds(j * CHUNK, CHUNK)],
                sem,
            )
            for j in range(BPW // CHUNK)
        ]
        for c in copies:
            c.wait()

        rot = lax.iota(jnp.int32, 16)
        eps2 = jnp.float32(1e-24)

        def body(g, acc):
            rows = g * L + rot
            sxw = jnp.zeros((L,), jnp.float32)
            sww = jnp.zeros((L,), jnp.float32)
            sxx = jnp.zeros((L,), jnp.float32)
            for f in range(D):
                fv = lax.bitwise_and(rot + f, D - 1)
                wv = plsc.load_gather(w_v, [rows, fv])
                xv = plsc.load_gather(x_v, [rows, fv])
                sxw = sxw + wv * xv
                sww = sww + wv * wv
                sxx = sxx + xv * xv
            rx = _rsqrt(jnp.maximum(sxx, eps2))
            rw = _rsqrt(jnp.maximum(sww, eps2))
            return acc + sxw * rx * rw

        acc = lax.fori_loop(0, GROUPS, body, jnp.zeros((L,), jnp.float32))
        acc_v[...] = acc
        pltpu.sync_copy(acc_v, out_hbm.at[wid])

    return k(x, target, weight)


def _finish(partials):
    def fk(p_ref, o_ref):
        s = -jnp.sum(p_ref[...]) * jnp.float32(1.0 / B)
        o_ref[...] = jnp.broadcast_to(s, (1, 1))

    r = pl.pallas_call(
        fk,
        out_shape=jax.ShapeDtypeStruct((1, 1), jnp.float32),
    )(partials)
    return r[0, 0]


def kernel(x, target, weight):
    partials = _sc_partials(x, target, weight)
    return _finish(partials.reshape(4, 128))


# speedup vs baseline: 2.6088x; 2.6088x over previous
import functools

import jax
import jax.numpy as jnp
from jax import lax
from jax.experimental import pallas as pl
from jax.experimental.pallas import tpu as pltpu
from jax.experimental.pallas import tpu_sc as plsc

B = 16384
D = 64
N = 1000000
NW = 32
L = 16
CW = 512
TPW = 244
NCH = 61
OUTR = B + 64
DUMP = B


def _sc_gather(wt, wtail, target):
    mesh = plsc.VectorSubcoreMesh(core_axis_name="c", subcore_axis_name="s")

    @functools.partial(
        pl.kernel, mesh=mesh,
        out_type=jax.ShapeDtypeStruct((OUTR, 128), jnp.float32),
        compiler_params=pltpu.CompilerParams(
            needs_layout_passes=False, use_tc_tiling_on_sc=True),
        scratch_types=[
            pltpu.VMEM((B,), jnp.int32),
            pltpu.VMEM((B + L,), jnp.int32),
            pltpu.VMEM((B + L,), jnp.int32),
            pltpu.VMEM((D, CW), jnp.float32),
            pltpu.VMEM((D, 64), jnp.float32),
            pltpu.VMEM((64, 128), jnp.float32),
            pltpu.VMEM((64,), jnp.int32),
            pltpu.SemaphoreType.DMA,
        ],
    )
    def k(wt_hbm, wtail_hbm, tgt_hbm, out_hbm, tgt_v, mlist, cbuf, chunk_v,
          tail_v, stage_v, posv, sem):
        wid = lax.axis_index("s") * 2 + lax.axis_index("c")
        base_tc = wid * TPW
        base_cls = base_tc * 128
        rot = lax.iota(jnp.int32, 16)

        pltpu.sync_copy(tgt_hbm, tgt_v)
        for s0 in range(0, 64, 16):
            posv[pl.ds(s0, 16)] = jnp.zeros((16,), jnp.int32) + DUMP

        def sel(i, n):
            t = tgt_v[pl.ds(i * L, L)]
            j = lax.shift_right_logical(t, 7)
            w_of = jnp.minimum(lax.div(j, TPW), NW - 1)
            m = w_of == wid
            kk = lax.shift_right_logical(j - base_tc, 2)
            rloc = lax.bitwise_and(t, CW - 1)
            pos = i * L + rot
            packed = lax.bitwise_or(
                lax.bitwise_or(lax.shift_left(kk, 23),
                               lax.shift_left(rloc, 14)), pos)
            plsc.store_compressed(mlist.at[pl.ds(n, L)], packed, mask=m)
            cnt = plsc.all_reduce_population_count(m)
            return n + lax.reduce_max(cnt, (0,))

        nmatch = lax.fori_loop(0, B // L, sel, jnp.int32(0))
        nvec = lax.div(nmatch + (L - 1), L)

        def do_chunk(buf_ref, kk, nslot):
            def csel(v, nc):
                e = mlist[pl.ds(v * L, L)]
                is_k = lax.shift_right_logical(e, 23) == kk
                in_rng = (v * L + rot) < nmatch
                sl = jnp.logical_and(is_k, in_rng)
                plsc.store_compressed(cbuf.at[pl.ds(nc, L)], e, mask=sl)
                c = plsc.all_reduce_population_count(sl)
                return nc + lax.reduce_max(c, (0,))

            nc = lax.fori_loop(0, nvec, csel, jnp.int32(0))

            def grp_body(g, ns):
                evec = cbuf[pl.ds(g * L, L)]

                def lane(l, ns):
                    def do():
                        e = evec[l]
                        rloc = lax.bitwise_and(
                            lax.shift_right_logical(e, 14), CW - 1)
                        pos = lax.bitwise_and(e, (1 << 14) - 1)
                        rb = jnp.zeros((L,), jnp.int32) + rloc
                        nsb = jnp.zeros((L,), jnp.int32) + ns
                        for q in range(4):
                            v = plsc.load_gather(buf_ref, [q * L + rot, rb])
                            plsc.store_scatter(stage_v, [nsb, q * L + rot], v)
                        plsc.store_scatter(
                            posv, [nsb], jnp.zeros((L,), jnp.int32) + pos,
                            mask=rot == 0)

                        def flush():
                            pltpu.async_copy(
                                stage_v, out_hbm.at[posv], sem).wait()

                        pl.when(ns == 63)(flush)

                    live = g * L + l < nc
                    pl.when(live)(do)
                    return lax.select(
                        jnp.logical_and(live, ns == 63), jnp.int32(0),
                        lax.select(live, ns + 1, ns))

                for l in range(L):
                    ns = lane(l, ns)
                return ns

            ngrp = lax.div(nc + (L - 1), L)
            return lax.fori_loop(0, ngrp, grp_body, nslot)

        def body(kk, nslot):
            pltpu.async_copy(
                wt_hbm.at[:, pl.ds(base_cls + kk * CW, CW)],
                chunk_v, sem).wait()
            return do_chunk(chunk_v, kk, nslot)

        nch = jnp.int32(NCH) + lax.select(wid == NW - 1, jnp.int32(1),
                                          jnp.int32(0))
        nslot = lax.fori_loop(0, nch, body, jnp.int32(0))

        def tail():
            pltpu.async_copy(wtail_hbm, tail_v, sem).wait()

        pl.when(wid == NW - 1)(tail)
        nslot2 = lax.select(
            wid == NW - 1, do_chunk(tail_v, jnp.int32(62), nslot), nslot)

        _ = nslot2
        pltpu.async_copy(stage_v, out_hbm.at[posv], sem).wait()

    return k(wt, wtail, target)


def _finisher(wg, x):
    BLK = 2048

    def fk(wg_ref, x_ref, o_ref):
        eps = jnp.float32(1e-12)
        i = pl.program_id(0)

        @pl.when(i == 0)
        def _():
            o_ref[...] = jnp.zeros((1, 1), jnp.float32)

        w = wg_ref[...][:, :D]
        xv = x_ref[...]
        sxw = jnp.sum(xv * w, axis=1)
        sww = jnp.sum(w * w, axis=1)
        sxx = jnp.sum(xv * xv, axis=1)
        nx = jnp.maximum(jnp.sqrt(sxx), eps)
        nw = jnp.maximum(jnp.sqrt(sww), eps)
        cos = sxw / (nx * nw)
        o_ref[...] = o_ref[...] + jnp.sum(cos) * jnp.float32(-1.0 / B)

    return pl.pallas_call(
        fk,
        grid=(B // BLK,),
        in_specs=[
            pl.BlockSpec((BLK, 128), lambda i: (i, 0)),
            pl.BlockSpec((BLK, D), lambda i: (i, 0)),
        ],
        out_specs=pl.BlockSpec((1, 1), lambda i: (0, 0)),
        out_shape=jax.ShapeDtypeStruct((1, 1), jnp.float32),
    )(wg, x)


def kernel(x, target, weight):
    wt = weight.T
    wtail = lax.slice(weight, (N - 64, 0), (N, D)).T
    wg = _sc_gather(wt, wtail, target)
    return _finisher(wg, x)[0, 0]


# speedup vs baseline: 3.8549x; 1.4776x over previous
import functools

import jax
import jax.numpy as jnp
from jax import lax
from jax.experimental import pallas as pl
from jax.experimental.pallas import tpu as pltpu
from jax.experimental.pallas import tpu_sc as plsc

B = 16384
D = 64
N = 1000000
NW = 32
L = 16
CW = 512
TPW = 244
NCH = 61
OUTR = B + 64
DUMP = B


def _sc_gather(wt, wtail, target):
    mesh = plsc.VectorSubcoreMesh(core_axis_name="c", subcore_axis_name="s")

    @functools.partial(
        pl.kernel, mesh=mesh,
        out_type=jax.ShapeDtypeStruct((OUTR, 128), jnp.float32),
        compiler_params=pltpu.CompilerParams(
            needs_layout_passes=False, use_tc_tiling_on_sc=True),
        scratch_types=[
            pltpu.VMEM((B + L,), jnp.int32),
            pltpu.VMEM((B + L,), jnp.int32),
            pltpu.VMEM((D, CW), jnp.float32),
            pltpu.VMEM((D, CW), jnp.float32),
            pltpu.VMEM((D, 64), jnp.float32),
            pltpu.VMEM((64, 128), jnp.float32),
            pltpu.VMEM((64,), jnp.int32),
            pltpu.SemaphoreType.DMA,
            pltpu.SemaphoreType.DMA,
            pltpu.SemaphoreType.DMA,
        ],
    )
    def k(wt_hbm, wtail_hbm, tgt_hbm, out_hbm, cbuf, mlist, chunk0, chunk1,
          tail_v, stage_v, posv, sem, sem0, sem1):
        wid = lax.axis_index("s") * 2 + lax.axis_index("c")
        base_tc = wid * TPW
        base_cls = base_tc * 128
        rot = lax.iota(jnp.int32, 16)

        pltpu.sync_copy(tgt_hbm, cbuf.at[pl.ds(0, B)])
        def fetch_tail():
            pltpu.async_copy(wtail_hbm, tail_v, sem).wait()

        pl.when(wid == NW - 1)(fetch_tail)
        for s0 in range(0, 64, 16):
            posv[pl.ds(s0, 16)] = jnp.zeros((16,), jnp.int32) + DUMP

        def sel(i, n):
            t = cbuf[pl.ds(i * L, L)]
            j = lax.shift_right_logical(t, 7)
            w_of = jnp.minimum(lax.div(j, TPW), NW - 1)
            m = w_of == wid
            kk = lax.shift_right_logical(j - base_tc, 2)
            rloc = lax.bitwise_and(t, CW - 1)
            pos = i * L + rot
            packed = lax.bitwise_or(
                lax.bitwise_or(lax.shift_left(kk, 23),
                               lax.shift_left(rloc, 14)), pos)
            plsc.store_compressed(mlist.at[pl.ds(n, L)], packed, mask=m)
            cnt = plsc.all_reduce_population_count(m)
            return n + lax.reduce_max(cnt, (0,))

        nmatch = lax.fori_loop(0, B // L, sel, jnp.int32(0))
        nvec = lax.div(nmatch + (L - 1), L)

        def do_chunk(buf_ref, kk, nslot):
            def csel(v, nc):
                e = mlist[pl.ds(v * L, L)]
                is_k = lax.shift_right_logical(e, 23) == kk
                in_rng = (v * L + rot) < nmatch
                sl = jnp.logical_and(is_k, in_rng)
                plsc.store_compressed(cbuf.at[pl.ds(nc, L)], e, mask=sl)
                c = plsc.all_reduce_population_count(sl)
                return nc + lax.reduce_max(c, (0,))

            nc = lax.fori_loop(0, nvec, csel, jnp.int32(0))

            def grp_body(g, ns):
                evec = cbuf[pl.ds(g * L, L)]

                def lane(l, ns):
                    def do():
                        e = evec[l]
                        rloc = lax.bitwise_and(
                            lax.shift_right_logical(e, 14), CW - 1)
                        pos = lax.bitwise_and(e, (1 << 14) - 1)
                        rb = jnp.zeros((L,), jnp.int32) + rloc
                        nsb = jnp.zeros((L,), jnp.int32) + ns
                        for q in range(4):
                            v = plsc.load_gather(buf_ref, [q * L + rot, rb])
                            plsc.store_scatter(stage_v, [nsb, q * L + rot], v)
                        plsc.store_scatter(
                            posv, [nsb], jnp.zeros((L,), jnp.int32) + pos,
                            mask=rot == 0)

                        def flush():
                            pltpu.async_copy(
                                stage_v, out_hbm.at[posv], sem).wait()

                        pl.when(ns == 63)(flush)

                    live = g * L + l < nc
                    pl.when(live)(do)
                    return lax.select(
                        jnp.logical_and(live, ns == 63), jnp.int32(0),
                        lax.select(live, ns + 1, ns))

                for l in range(L):
                    ns = lane(l, ns)
                return ns

            ngrp = lax.div(nc + (L - 1), L)
            return lax.fori_loop(0, ngrp, grp_body, nslot)

        nch = jnp.int32(NCH) + lax.select(wid == NW - 1, jnp.int32(1),
                                          jnp.int32(0))

        def src(kk):
            return wt_hbm.at[:, pl.ds(base_cls + kk * CW, CW)]

        _ = pltpu.async_copy(src(jnp.int32(0)), chunk0, sem0)

        def pair_body(m, nslot):
            c0 = 2 * m
            c1 = c0 + 1
            def start1():
                pltpu.async_copy(src(c1), chunk1, sem1)

            pl.when(c1 < nch)(start1)
            pltpu.make_async_copy(src(c0), chunk0, sem0).wait()
            nslot = do_chunk(chunk0, c0, nslot)
            def start0():
                pltpu.async_copy(src(c0 + 2), chunk0, sem0)

            pl.when(c0 + 2 < nch)(start0)

            def second(ns):
                pltpu.make_async_copy(src(c1), chunk1, sem1).wait()
                return do_chunk(chunk1, c1, ns)

            return lax.cond(c1 < nch, second, lambda ns: ns, nslot)

        nslot = lax.fori_loop(0, (NCH + 2) // 2, pair_body, jnp.int32(0))

        nslot2 = lax.select(
            wid == NW - 1, do_chunk(tail_v, jnp.int32(62), nslot), nslot)

        _ = nslot2
        pltpu.async_copy(stage_v, out_hbm.at[posv], sem).wait()

    return k(wt, wtail, target)


def _finisher(wg, x):
    BLK = 2048

    def fk(wg_ref, x_ref, o_ref):
        eps = jnp.float32(1e-12)
        i = pl.program_id(0)

        @pl.when(i == 0)
        def _():
            o_ref[...] = jnp.zeros((1, 1), jnp.float32)

        w = wg_ref[...][:, :D]
        xv = x_ref[...]
        sxw = jnp.sum(xv * w, axis=1)
        sww = jnp.sum(w * w, axis=1)
        sxx = jnp.sum(xv * xv, axis=1)
        nx = jnp.maximum(jnp.sqrt(sxx), eps)
        nw = jnp.maximum(jnp.sqrt(sww), eps)
        cos = sxw / (nx * nw)
        o_ref[...] = o_ref[...] + jnp.sum(cos) * jnp.float32(-1.0 / B)

    return pl.pallas_call(
        fk,
        grid=(B // BLK,),
        in_specs=[
            pl.BlockSpec((BLK, 128), lambda i: (i, 0)),
            pl.BlockSpec((BLK, D), lambda i: (i, 0)),
        ],
        out_specs=pl.BlockSpec((1, 1), lambda i: (0, 0)),
        out_shape=jax.ShapeDtypeStruct((1, 1), jnp.float32),
    )(wg, x)


def kernel(x, target, weight):
    wt = weight.T
    wtail = lax.slice(weight, (N - 64, 0), (N, D)).T
    wg = _sc_gather(wt, wtail, target)
    return _finisher(wg, x)[0, 0]
